# baseline (speedup 1.0000x reference)
import functools

import jax
import jax.numpy as jnp
from jax import lax
from jax.experimental import pallas as pl
from jax.experimental.pallas import tpu as pltpu
from jax.experimental.pallas import tpu_sc as plsc

_D = 128
_CHUNK = 128
_K = 5


@functools.lru_cache(maxsize=None)
def _make_gather(B: int):
    info = plsc.get_sparse_core_info()
    nw = info.num_cores * info.num_subcores
    b_per_w = B // nw
    n_chunks = b_per_w // _CHUNK
    n_groups = n_chunks // _K
    mesh = plsc.VectorSubcoreMesh(core_axis_name="c", subcore_axis_name="s")

    @functools.partial(
        pl.kernel,
        mesh=mesh,
        out_type=jax.ShapeDtypeStruct((B, _D), jnp.float32),
        scratch_types=[
            pltpu.VMEM((n_chunks, _CHUNK), jnp.int32),
        ]
        + [pltpu.VMEM((_CHUNK, _D), jnp.float32) for _ in range(_K)]
        + [pltpu.SemaphoreType.DMA for _ in range(2 * _K)],
    )
    def gather(idx_hbm, table_hbm, out_hbm, idx_v, *bufs_and_sems):
        bufs = bufs_and_sems[:_K]
        gsem = bufs_and_sems[_K : 2 * _K]
        ssem = bufs_and_sems[2 * _K :]
        wid = lax.axis_index("s") * info.num_cores + lax.axis_index("c")
        base = wid * b_per_w
        pltpu.sync_copy(idx_hbm.at[wid], idx_v)

        def start_gather(b, j):
            pltpu.async_copy(table_hbm.at[idx_v.at[j]], bufs[b], gsem[b])

        def wait_gather(b, j):
            pltpu.make_async_copy(table_hbm.at[idx_v.at[j]], bufs[b], gsem[b]).wait()

        def start_store(b, j):
            pltpu.async_copy(
                bufs[b], out_hbm.at[pl.ds(base + j * _CHUNK, _CHUNK)], ssem[b]
            )

        def wait_store(b, j):
            pltpu.make_async_copy(
                bufs[b], out_hbm.at[pl.ds(base + j * _CHUNK, _CHUNK)], ssem[b]
            ).wait()

        for c in range(_K - 2):
            start_gather(c, c)
        for b in range(_K):
            wait_gather(b, b)
            start_store(b, b)
            c = b + _K - 2
            if c < n_chunks:
                if b >= 2:
                    wait_store((b - 2) % _K, b - 2)
                start_gather((b - 2) % _K, c)

        def body(g, carry):
            j0 = g * _K
            for b in range(_K):
                j = j0 + b
                wait_gather(b, j)
                start_store(b, j)
                bc = (b - 2) % _K

                @pl.when(j + _K - 2 < n_chunks)
                def _prefetch():
                    wait_store(bc, j - 2)
                    start_gather(bc, j + _K - 2)

            return carry

        lax.fori_loop(1, n_groups, body, 0)
        for b in range(_K):
            wait_store(b, (n_groups - 1) * _K + b)

    return gather, nw, n_chunks


def kernel(token_ids, weight):
    S, T = token_ids.shape
    B = S * T
    gather, nw, n_chunks = _make_gather(B)
    idx = token_ids.T.reshape(nw, n_chunks, _CHUNK).astype(jnp.int32)
    out = gather(idx, weight)
    return out.reshape(T, S, _D).transpose(1, 0, 2)


# speedup vs baseline: 1.0003x; 1.0003x over previous
import functools

import jax
import jax.numpy as jnp
from jax import lax
from jax.experimental import pallas as pl
from jax.experimental.pallas import tpu as pltpu
from jax.experimental.pallas import tpu_sc as plsc

_D = 128
_CHUNK = 64
_K = 10


@functools.lru_cache(maxsize=None)
def _make_gather(B: int):
    info = plsc.get_sparse_core_info()
    nw = info.num_cores * info.num_subcores
    b_per_w = B // nw
    n_chunks = b_per_w // _CHUNK
    n_groups = n_chunks // _K
    mesh = plsc.VectorSubcoreMesh(core_axis_name="c", subcore_axis_name="s")

    @functools.partial(
        pl.kernel,
        mesh=mesh,
        out_type=jax.ShapeDtypeStruct((B, _D), jnp.float32),
        scratch_types=[
            pltpu.VMEM((n_chunks, _CHUNK), jnp.int32),
        ]
        + [pltpu.VMEM((_CHUNK, _D), jnp.float32) for _ in range(_K)]
        + [pltpu.SemaphoreType.DMA for _ in range(2 * _K)],
    )
    def gather(idx_hbm, table_hbm, out_hbm, idx_v, *bufs_and_sems):
        bufs = bufs_and_sems[:_K]
        gsem = bufs_and_sems[_K : 2 * _K]
        ssem = bufs_and_sems[2 * _K :]
        wid = lax.axis_index("s") * info.num_cores + lax.axis_index("c")
        base = wid * b_per_w
        pltpu.sync_copy(idx_hbm.at[wid], idx_v)

        def start_gather(b, j):
            pltpu.async_copy(table_hbm.at[idx_v.at[j]], bufs[b], gsem[b])

        def wait_gather(b, j):
            pltpu.make_async_copy(table_hbm.at[idx_v.at[j]], bufs[b], gsem[b]).wait()

        def start_store(b, j):
            pltpu.async_copy(
                bufs[b], out_hbm.at[pl.ds(base + j * _CHUNK, _CHUNK)], ssem[b]
            )

        def wait_store(b, j):
            pltpu.make_async_copy(
                bufs[b], out_hbm.at[pl.ds(base + j * _CHUNK, _CHUNK)], ssem[b]
            ).wait()

        for c in range(_K - 2):
            start_gather(c, c)
        for b in range(_K):
            wait_gather(b, b)
            start_store(b, b)
            c = b + _K - 2
            if c < n_chunks:
                if b >= 2:
                    wait_store((b - 2) % _K, b - 2)
                start_gather((b - 2) % _K, c)

        def body(g, carry):
            j0 = g * _K
            for b in range(_K):
                j = j0 + b
                wait_gather(b, j)
                start_store(b, j)
                bc = (b - 2) % _K

                @pl.when(j + _K - 2 < n_chunks)
                def _prefetch():
                    wait_store(bc, j - 2)
                    start_gather(bc, j + _K - 2)

            return carry

        lax.fori_loop(1, n_groups, body, 0)
        for b in range(_K):
            wait_store(b, (n_groups - 1) * _K + b)

    return gather, nw, n_chunks


def kernel(token_ids, weight):
    S, T = token_ids.shape
    B = S * T
    gather, nw, n_chunks = _make_gather(B)
    idx = token_ids.T.reshape(nw, n_chunks, _CHUNK).astype(jnp.int32)
    out = gather(idx, weight)
    return out.reshape(T, S, _D).transpose(1, 0, 2)
